# baseline (speedup 1.0000x reference)
import functools

import jax
import jax.numpy as jnp
from jax.experimental import pallas as pl
from jax.experimental.pallas import tpu as pltpu

_BN_EPS = 1e-5


def _conv_stats_kernel(x_ref, w1_ref, y_ref, st_ref, xw_ref, *, H, W, Cin,
                       C1):
    HW = H * W
    x = x_ref[0]
    for kx in range(3):
        xw_ref[:, :, kx * Cin:(kx + 1) * Cin] = (
            x[:, kx:kx + W, :].astype(jnp.bfloat16))
    acc = jnp.zeros((C1, HW), jnp.float32)
    for ky in range(3):
        tap = xw_ref[ky:ky + H].reshape(HW, 3 * Cin)
        acc = acc + jax.lax.dot_general(
            w1_ref[ky], tap, (((0,), (1,)), ((), ())),
            preferred_element_type=jnp.float32)
    y_ref[0] = acc
    st_ref[0, :, 0:1] = jnp.sum(acc, axis=1, keepdims=True)
    st_ref[0, :, 1:2] = jnp.sum(acc * acc, axis=1, keepdims=True)


def _head_kernel(y_ref, ss_ref, w2_ref, b2_ref, out_ref):
    y = y_ref[0]
    z = jnp.maximum(y * ss_ref[:, 0:1] + ss_ref[:, 1:2], 0.0)
    z = z.astype(jnp.bfloat16)
    out = jnp.dot(w2_ref[...], z, preferred_element_type=jnp.float32)
    out_ref[0] = out + b2_ref[...]


def kernel(x_nchw, w1, b1, gamma, beta, w2, b2):
    del b1
    N, Cin, H, W = x_nchw.shape
    C1 = w1.shape[-1]
    O = w2.shape[-1]
    HW = H * W
    rows = N * HW

    x_pad = jnp.pad(
        jnp.transpose(x_nchw, (0, 2, 3, 1)),
        ((0, 0), (1, 1), (1, 1), (0, 0)))
    w1b = w1.reshape(3, 3 * Cin, C1).astype(jnp.bfloat16)
    w2t = w2.reshape(C1, O).T.astype(jnp.bfloat16)
    b2c = b2.reshape(O, 1).astype(jnp.float32)

    cparams = pltpu.CompilerParams(
        dimension_semantics=("arbitrary",),
        vmem_limit_bytes=48 * 1024 * 1024,
    )

    conv_flops = 2 * rows * 9 * Cin * C1
    y, stats = pl.pallas_call(
        functools.partial(_conv_stats_kernel, H=H, W=W, Cin=Cin, C1=C1),
        out_shape=(jax.ShapeDtypeStruct((N, C1, HW), jnp.float32),
                   jax.ShapeDtypeStruct((N, C1, 2), jnp.float32)),
        grid=(N,),
        in_specs=[pl.BlockSpec((1, H + 2, W + 2, Cin), lambda n: (n, 0, 0, 0)),
                  pl.BlockSpec((3, 3 * Cin, C1), lambda n: (0, 0, 0))],
        out_specs=(pl.BlockSpec((1, C1, HW), lambda n: (n, 0, 0)),
                   pl.BlockSpec((1, C1, 2), lambda n: (n, 0, 0))),
        scratch_shapes=[pltpu.VMEM((H + 2, W, 3 * Cin), jnp.bfloat16)],
        compiler_params=cparams,
        cost_estimate=pl.CostEstimate(
            flops=conv_flops, transcendentals=0,
            bytes_accessed=x_pad.size * 4 + w1b.size * 2
            + (rows + 2 * N) * C1 * 4),
    )(x_pad, w1b)

    mean = jnp.sum(stats[:, :, 0], axis=0) * (1.0 / rows)
    var = jnp.maximum(
        jnp.sum(stats[:, :, 1], axis=0) * (1.0 / rows) - mean * mean, 0.0)
    scale = gamma.reshape(C1) * jax.lax.rsqrt(var + _BN_EPS)
    shift = beta.reshape(C1) - mean * scale
    ss = jnp.stack([scale, shift], axis=1)

    out = pl.pallas_call(
        _head_kernel,
        out_shape=jax.ShapeDtypeStruct((N, O, HW), jnp.float32),
        grid=(N,),
        in_specs=[pl.BlockSpec((1, C1, HW), lambda n: (n, 0, 0)),
                  pl.BlockSpec((C1, 2), lambda n: (0, 0)),
                  pl.BlockSpec((O, C1), lambda n: (0, 0)),
                  pl.BlockSpec((O, 1), lambda n: (0, 0))],
        out_specs=pl.BlockSpec((1, O, HW), lambda n: (n, 0, 0)),
        compiler_params=cparams,
        cost_estimate=pl.CostEstimate(
            flops=2 * rows * C1 * O, transcendentals=0,
            bytes_accessed=rows * C1 * 4 + w2t.size * 2 + rows * O * 4),
    )(y, ss, w2t, b2c)

    return out.reshape(N, O, H, W)


# speedup vs baseline: 1.5401x; 1.5401x over previous
import functools

import jax
import jax.numpy as jnp
from jax.experimental import pallas as pl
from jax.experimental.pallas import tpu as pltpu

_BN_EPS = 1e-5


def _conv_stats_kernel(x_ref, w1_ref, y_ref, st_ref, xw_ref, *, H, W, Cin,
                       C1):
    HW = H * W
    x = x_ref[0]
    for kx in range(3):
        xw_ref[:, :, kx * Cin:(kx + 1) * Cin] = (
            x[:, kx:kx + W, :].astype(jnp.bfloat16))
    acc = jnp.zeros((C1, HW), jnp.float32)
    for ky in range(3):
        tap = xw_ref[ky:ky + H].reshape(HW, 3 * Cin)
        acc = acc + jax.lax.dot_general(
            w1_ref[ky], tap, (((0,), (1,)), ((), ())),
            preferred_element_type=jnp.float32)
    y_ref[0] = acc
    st_ref[0, :, 0:1] = jnp.sum(acc, axis=1, keepdims=True)
    st_ref[0, :, 1:2] = jnp.sum(acc * acc, axis=1, keepdims=True)


def _head_kernel(y_ref, ss_ref, w2_ref, b2_ref, out_ref):
    y = y_ref[0]
    z = jnp.maximum(y * ss_ref[:, 0:1] + ss_ref[:, 1:2], 0.0)
    z = z.astype(jnp.bfloat16)
    out = jax.lax.dot_general(z, w2_ref[...], (((0,), (0,)), ((), ())),
                              preferred_element_type=jnp.float32)
    out_ref[0] = out + b2_ref[...]


def kernel(x_nchw, w1, b1, gamma, beta, w2, b2):
    del b1
    N, Cin, H, W = x_nchw.shape
    C1 = w1.shape[-1]
    O = w2.shape[-1]
    HW = H * W
    rows = N * HW

    x_pad = jnp.pad(
        jnp.transpose(x_nchw, (0, 2, 3, 1)),
        ((0, 0), (1, 1), (1, 1), (0, 0)))
    w1b = w1.reshape(3, 3 * Cin, C1).astype(jnp.bfloat16)
    w2b = w2.reshape(C1, O).astype(jnp.bfloat16)
    b2c = b2.reshape(1, O).astype(jnp.float32)

    cparams = pltpu.CompilerParams(
        dimension_semantics=("arbitrary",),
        vmem_limit_bytes=48 * 1024 * 1024,
    )

    conv_flops = 2 * rows * 9 * Cin * C1
    y, stats = pl.pallas_call(
        functools.partial(_conv_stats_kernel, H=H, W=W, Cin=Cin, C1=C1),
        out_shape=(jax.ShapeDtypeStruct((N, C1, HW), jnp.float32),
                   jax.ShapeDtypeStruct((N, C1, 2), jnp.float32)),
        grid=(N,),
        in_specs=[pl.BlockSpec((1, H + 2, W + 2, Cin), lambda n: (n, 0, 0, 0)),
                  pl.BlockSpec((3, 3 * Cin, C1), lambda n: (0, 0, 0))],
        out_specs=(pl.BlockSpec((1, C1, HW), lambda n: (n, 0, 0)),
                   pl.BlockSpec((1, C1, 2), lambda n: (n, 0, 0))),
        scratch_shapes=[pltpu.VMEM((H + 2, W, 3 * Cin), jnp.bfloat16)],
        compiler_params=cparams,
        cost_estimate=pl.CostEstimate(
            flops=conv_flops, transcendentals=0,
            bytes_accessed=x_pad.size * 4 + w1b.size * 2
            + (rows + 2 * N) * C1 * 4),
    )(x_pad, w1b)

    mean = jnp.sum(stats[:, :, 0], axis=0) * (1.0 / rows)
    var = jnp.maximum(
        jnp.sum(stats[:, :, 1], axis=0) * (1.0 / rows) - mean * mean, 0.0)
    scale = gamma.reshape(C1) * jax.lax.rsqrt(var + _BN_EPS)
    shift = beta.reshape(C1) - mean * scale
    ss = jnp.stack([scale, shift], axis=1)

    out = pl.pallas_call(
        _head_kernel,
        out_shape=jax.ShapeDtypeStruct((N, HW, O), jnp.float32),
        grid=(N,),
        in_specs=[pl.BlockSpec((1, C1, HW), lambda n: (n, 0, 0)),
                  pl.BlockSpec((C1, 2), lambda n: (0, 0)),
                  pl.BlockSpec((C1, O), lambda n: (0, 0)),
                  pl.BlockSpec((1, O), lambda n: (0, 0))],
        out_specs=pl.BlockSpec((1, HW, O), lambda n: (n, 0, 0)),
        compiler_params=cparams,
        cost_estimate=pl.CostEstimate(
            flops=2 * rows * C1 * O, transcendentals=0,
            bytes_accessed=rows * C1 * 4 + w2b.size * 2 + rows * O * 4),
    )(y, ss, w2b, b2c)

    out = out.reshape(N, H, W, O)
    return jnp.transpose(out, (0, 3, 1, 2))
